# Initial kernel scaffold; baseline (speedup 1.0000x reference)
import jax
import jax.numpy as jnp
from jax.experimental import pallas as pl


def kernel(x, edge_index, edge_attr, batch, atom_table, vn_table, eps, bond_tables, conv_w1, conv_b1, conv_ln_g, conv_ln_b, conv_w2, conv_b2, norm_g, norm_b, vn_w1, vn_b1, vn_ln_g, vn_ln_b, vn_w2, vn_b2):
    raise NotImplementedError("write your pallas kernel here")


# speedup vs baseline: 7.1956x; 7.1956x over previous
import functools

import jax
import jax.numpy as jnp
from jax import lax
from jax.experimental import pallas as pl
from jax.experimental.pallas import tpu as pltpu
from jax.experimental.pallas import tpu_sc as plsc

N = 10000
E = 320000
L = 3
H = 128
FF = 512
NG = 64
NA = 5
R = 1000
NB = N // R
NEG = -1e30

SC_NC = 2
SC_NS = 16
SC_NW = SC_NC * SC_NS
EPW = E // SC_NW
CH = 80
NCH = EPW // CH
NP = 10240
RPS = NP // SC_NS


def _ln(t, g, b, eps=1e-5):
    m = jnp.mean(t, axis=-1, keepdims=True)
    v = jnp.mean((t - m) ** 2, axis=-1, keepdims=True)
    return (t - m) * lax.rsqrt(v + eps) * g + b


def _gelu(t):
    return 0.5 * t * (1.0 + lax.erf(t * (2.0 ** -0.5)))


def _dotT(a, b):
    return lax.dot_general(a, b, (((1,), (1,)), ((), ())),
                           preferred_element_type=jnp.float32)


def _onehot_t(ids, ncls):
    return (lax.broadcasted_iota(jnp.int32, (ncls, 1), 0) == ids
            ).astype(jnp.float32)


def _embed_body(x_ref, tab_ref, out_ref):
    ids = x_ref[...].reshape(1, R)
    oht = _onehot_t(ids, 128)
    out_ref[...] = lax.dot_general(oht, tab_ref[...], (((0,), (0,)), ((), ())),
                                   preferred_element_type=jnp.float32)


def _embed(x3, atom_pad):
    return pl.pallas_call(
        _embed_body,
        grid=(NB,),
        in_specs=[pl.BlockSpec((1, 1, R), lambda i: (i, 0, 0)),
                  pl.BlockSpec((128, H), lambda i: (0, 0))],
        out_specs=pl.BlockSpec((R, H), lambda i: (i, 0)),
        out_shape=jax.ShapeDtypeStruct((N, H), jnp.float32),
    )(x3, atom_pad)


def _stage_a_body(h_ref, b_ref, vn_ref, bond_ref, eye_ref,
                  hl_ref, g_ref, vp_ref, *, with_vpool):
    ids = b_ref[...].reshape(1, R)
    oht = _onehot_t(ids, NG)
    hl = h_ref[...] + lax.dot_general(
        oht, vn_ref[...], (((0,), (0,)), ((), ())),
        preferred_element_type=jnp.float32)
    hl_ref[...] = hl
    for a in range(NA):
        g_ref[a] = _gelu(hl + bond_ref[a:a + 1, :])
    if with_vpool:
        oh = lax.dot_general(oht, eye_ref[...], (((0,), (0,)), ((), ())),
                             preferred_element_type=jnp.float32)
        pen = (oh - 1.0) * 1e30

        @pl.when(pl.program_id(0) == 0)
        def _():
            vp_ref[...] = jnp.full((NG, H), NEG, jnp.float32)

        for g in range(NG):
            cand = jnp.max(hl + pen[:, g:g + 1], axis=0, keepdims=True)
            vp_ref[g:g + 1, :] = jnp.maximum(vp_ref[g:g + 1, :], cand)


def _stage_a(h, batch3, vn, bond, eye, with_vpool):
    body = functools.partial(_stage_a_body, with_vpool=with_vpool)
    return pl.pallas_call(
        body,
        grid=(NB,),
        in_specs=[pl.BlockSpec((R, H), lambda i: (i, 0)),
                  pl.BlockSpec((1, 1, R), lambda i: (i, 0, 0)),
                  pl.BlockSpec((NG, H), lambda i: (0, 0)),
                  pl.BlockSpec((NA, H), lambda i: (0, 0)),
                  pl.BlockSpec((NG, NG), lambda i: (0, 0))],
        out_specs=[pl.BlockSpec((R, H), lambda i: (i, 0)),
                   pl.BlockSpec((NA, R, H), lambda i: (0, i, 0)),
                   pl.BlockSpec((NG, H), lambda i: (0, 0))],
        out_shape=[jax.ShapeDtypeStruct((N, H), jnp.float32),
                   jax.ShapeDtypeStruct((NA, N, H), jnp.float32),
                   jax.ShapeDtypeStruct((NG, H), jnp.float32)],
    )(h, batch3, vn, bond, eye)


def _edge_aggr(gtab, gidx, didx, zeros):
    mesh = plsc.VectorSubcoreMesh(core_axis_name="c", subcore_axis_name="s")

    @functools.partial(
        pl.kernel,
        out_type=jax.ShapeDtypeStruct((SC_NC, NP, H), jnp.float32),
        mesh=mesh,
        scratch_types=[
            pltpu.VMEM((NCH, CH), jnp.int32),
            pltpu.VMEM((NCH, CH), jnp.int32),
            pltpu.VMEM((CH, H), jnp.float32),
            pltpu.VMEM_SHARED((NP, H), jnp.float32),
            pltpu.SemaphoreType.DMA,
        ],
    )
    def k(gtab_hbm, gidx_hbm, didx_hbm, zeros_hbm, out_hbm,
          gidx_v, didx_v, rows_v, aggr_sh, sem):
        cid = lax.axis_index("c")
        sid = lax.axis_index("s")
        wid = sid * SC_NC + cid
        pltpu.sync_copy(zeros_hbm.at[pl.ds(sid * RPS, RPS)],
                        aggr_sh.at[pl.ds(sid * RPS, RPS)])
        pltpu.sync_copy(gidx_hbm.at[wid], gidx_v)
        pltpu.sync_copy(didx_hbm.at[wid], didx_v)
        plsc.subcore_barrier()

        def body(j, carry):
            pltpu.async_copy(gtab_hbm.at[gidx_v.at[j]], rows_v, sem).wait()
            pltpu.sync_copy(rows_v, aggr_sh.at[didx_v.at[j]], add=True)
            return carry

        lax.fori_loop(0, NCH, body, 0)
        plsc.subcore_barrier()
        pltpu.sync_copy(aggr_sh.at[pl.ds(sid * RPS, RPS)],
                        out_hbm.at[cid, pl.ds(sid * RPS, RPS)])

    return k(gtab, gidx, didx, zeros)


def _stage_b_body(hl_ref, ag_ref, eps_ref, w1_ref, b1_ref, lg_ref, lb_ref,
                  w2_ref, b2_ref, ng_ref, nb_ref, out_ref, *, last):
    hl = hl_ref[...]
    t = (1.0 + eps_ref[0, 0]) * hl + ag_ref[0] + ag_ref[1]
    t = _dotT(t, w1_ref[...]) + b1_ref[...]
    t = _ln(t, lg_ref[...], lb_ref[...])
    t = _gelu(t)
    t = _dotT(t, w2_ref[...]) + b2_ref[...]
    h = _ln(t, ng_ref[...], nb_ref[...])
    if not last:
        h = _gelu(h)
    out_ref[...] = h + hl


def _stage_b(hl, aggr2, eps_l, w1, b1, lg, lb, w2, b2, ng, nb, last):
    body = functools.partial(_stage_b_body, last=last)
    return pl.pallas_call(
        body,
        grid=(NB,),
        in_specs=[pl.BlockSpec((R, H), lambda i: (i, 0)),
                  pl.BlockSpec((SC_NC, R, H), lambda i: (0, i, 0)),
                  pl.BlockSpec((1, 1), lambda i: (0, 0)),
                  pl.BlockSpec((FF, H), lambda i: (0, 0)),
                  pl.BlockSpec((1, FF), lambda i: (0, 0)),
                  pl.BlockSpec((1, FF), lambda i: (0, 0)),
                  pl.BlockSpec((1, FF), lambda i: (0, 0)),
                  pl.BlockSpec((H, FF), lambda i: (0, 0)),
                  pl.BlockSpec((1, H), lambda i: (0, 0)),
                  pl.BlockSpec((1, H), lambda i: (0, 0)),
                  pl.BlockSpec((1, H), lambda i: (0, 0))],
        out_specs=pl.BlockSpec((R, H), lambda i: (i, 0)),
        out_shape=jax.ShapeDtypeStruct((N, H), jnp.float32),
    )(hl, aggr2, eps_l, w1, b1, lg, lb, w2, b2, ng, nb)


def _vn_body(vp_ref, vn_ref, w1_ref, b1_ref, lg_ref, lb_ref, w2_ref, b2_ref,
             out_ref):
    u = _dotT(vp_ref[...], w1_ref[...]) + b1_ref[...]
    u = _ln(u, lg_ref[...], lb_ref[...])
    u = _gelu(u)
    u = _dotT(u, w2_ref[...]) + b2_ref[...]
    out_ref[...] = vn_ref[...] + u


def _vn_update(vpool, vn, w1, b1, lg, lb, w2, b2):
    return pl.pallas_call(
        _vn_body,
        out_shape=jax.ShapeDtypeStruct((NG, H), jnp.float32),
    )(vpool, vn, w1, b1, lg, lb, w2, b2)


def _segsum_body(h_ref, b_ref, out_ref):
    ids = b_ref[...].reshape(1, R)
    oht = _onehot_t(ids, NG)

    @pl.when(pl.program_id(0) == 0)
    def _():
        out_ref[...] = jnp.zeros((NG, H), jnp.float32)

    out_ref[...] += lax.dot_general(oht, h_ref[...], (((1,), (0,)), ((), ())),
                                    preferred_element_type=jnp.float32)


def _segsum(h, batch3):
    return pl.pallas_call(
        _segsum_body,
        grid=(NB,),
        in_specs=[pl.BlockSpec((R, H), lambda i: (i, 0)),
                  pl.BlockSpec((1, 1, R), lambda i: (i, 0, 0))],
        out_specs=pl.BlockSpec((NG, H), lambda i: (0, 0)),
        out_shape=jax.ShapeDtypeStruct((NG, H), jnp.float32),
    )(h, batch3)


def kernel(x, edge_index, edge_attr, batch, atom_table, vn_table, eps,
           bond_tables, conv_w1, conv_b1, conv_ln_g, conv_ln_b, conv_w2,
           conv_b2, norm_g, norm_b, vn_w1, vn_b1, vn_ln_g, vn_ln_b, vn_w2,
           vn_b2):
    f32 = jnp.float32
    x3 = x.astype(jnp.int32).reshape(NB, 1, R)
    batch3 = batch.astype(jnp.int32).reshape(NB, 1, R)
    atom_pad = jnp.concatenate(
        [atom_table.astype(f32),
         jnp.zeros((128 - atom_table.shape[0], H), f32)], axis=0)
    eye = jnp.eye(NG, dtype=f32)
    gidx = (edge_attr.astype(jnp.int32) * N
            + edge_index[0].astype(jnp.int32)).reshape(SC_NW, NCH, CH)
    didx = edge_index[1].astype(jnp.int32).reshape(SC_NW, NCH, CH)
    zeros = jnp.zeros((NP, H), f32)
    vn = jnp.tile(vn_table.astype(f32), (NG, 1))

    h = _embed(x3, atom_pad)
    for l in range(L):
        hl, gt, vpool = _stage_a(h, batch3, vn, bond_tables[l].astype(f32),
                                 eye, with_vpool=(l < L - 1))
        aggr2 = _edge_aggr(gt.reshape(NA * N, H), gidx, didx, zeros)
        if l < L - 1:
            vn = _vn_update(vpool, vn,
                            vn_w1[l].astype(f32), vn_b1[l].reshape(1, FF),
                            vn_ln_g[l].reshape(1, FF), vn_ln_b[l].reshape(1, FF),
                            vn_w2[l].astype(f32), vn_b2[l].reshape(1, H))
        h = _stage_b(hl, aggr2, eps[l].reshape(1, 1).astype(f32),
                     conv_w1[l].astype(f32), conv_b1[l].reshape(1, FF),
                     conv_ln_g[l].reshape(1, FF), conv_ln_b[l].reshape(1, FF),
                     conv_w2[l].astype(f32), conv_b2[l].reshape(1, H),
                     norm_g[l].reshape(1, H), norm_b[l].reshape(1, H),
                     last=(l == L - 1))
    return _segsum(h, batch3)


# speedup vs baseline: 9.6575x; 1.3421x over previous
import functools

import jax
import jax.numpy as jnp
from jax import lax
from jax.experimental import pallas as pl
from jax.experimental.pallas import tpu as pltpu
from jax.experimental.pallas import tpu_sc as plsc

N = 10000
E = 320000
L = 3
H = 128
FF = 512
NG = 64
NA = 5
R = 1000
NB = N // R
NEG = -1e30

SC_NC = 2
SC_NS = 16
SC_NW = SC_NC * SC_NS
EPW = E // SC_NW
CH = 80
NCH = EPW // CH
S = 25
NSTG = NCH // S
NP = 10240
RPS = NP // SC_NS


def _ln(t, g, b, eps=1e-5):
    m = jnp.mean(t, axis=-1, keepdims=True)
    v = jnp.mean((t - m) ** 2, axis=-1, keepdims=True)
    return (t - m) * lax.rsqrt(v + eps) * g + b


def _gelu(t):
    return 0.5 * t * (1.0 + lax.erf(t * (2.0 ** -0.5)))


def _dotT(a, b):
    return lax.dot_general(a, b, (((1,), (1,)), ((), ())),
                           preferred_element_type=jnp.float32)


def _onehot_t(ids, ncls):
    return (lax.broadcasted_iota(jnp.int32, (ncls, 1), 0) == ids
            ).astype(jnp.float32)


def _embed_body(x_ref, tab_ref, out_ref):
    ids = x_ref[...].reshape(1, R)
    oht = _onehot_t(ids, 128)
    out_ref[...] = lax.dot_general(oht, tab_ref[...], (((0,), (0,)), ((), ())),
                                   preferred_element_type=jnp.float32)


def _embed(x3, atom_pad):
    return pl.pallas_call(
        _embed_body,
        grid=(NB,),
        in_specs=[pl.BlockSpec((1, 1, R), lambda i: (i, 0, 0)),
                  pl.BlockSpec((128, H), lambda i: (0, 0))],
        out_specs=pl.BlockSpec((R, H), lambda i: (i, 0)),
        out_shape=jax.ShapeDtypeStruct((N, H), jnp.float32),
    )(x3, atom_pad)


def _stage_a_body(h_ref, b_ref, vn_ref, bond_ref, eye_ref,
                  hl_ref, g_ref, vp_ref, *, with_vpool):
    ids = b_ref[...].reshape(1, R)
    oht = _onehot_t(ids, NG)
    hl = h_ref[...] + lax.dot_general(
        oht, vn_ref[...], (((0,), (0,)), ((), ())),
        preferred_element_type=jnp.float32)
    hl_ref[...] = hl
    for a in range(NA):
        g_ref[a] = _gelu(hl + bond_ref[a:a + 1, :])
    if with_vpool:
        oh = lax.dot_general(oht, eye_ref[...], (((0,), (0,)), ((), ())),
                             preferred_element_type=jnp.float32)
        pen = (oh - 1.0) * 1e30

        @pl.when(pl.program_id(0) == 0)
        def _():
            vp_ref[...] = jnp.full((NG, H), NEG, jnp.float32)

        for g in range(NG):
            cand = jnp.max(hl + pen[:, g:g + 1], axis=0, keepdims=True)
            vp_ref[g:g + 1, :] = jnp.maximum(vp_ref[g:g + 1, :], cand)


def _stage_a(h, batch3, vn, bond, eye, with_vpool):
    body = functools.partial(_stage_a_body, with_vpool=with_vpool)
    return pl.pallas_call(
        body,
        grid=(NB,),
        in_specs=[pl.BlockSpec((R, H), lambda i: (i, 0)),
                  pl.BlockSpec((1, 1, R), lambda i: (i, 0, 0)),
                  pl.BlockSpec((NG, H), lambda i: (0, 0)),
                  pl.BlockSpec((NA, H), lambda i: (0, 0)),
                  pl.BlockSpec((NG, NG), lambda i: (0, 0))],
        out_specs=[pl.BlockSpec((R, H), lambda i: (i, 0)),
                   pl.BlockSpec((NA, R, H), lambda i: (0, i, 0)),
                   pl.BlockSpec((NG, H), lambda i: (0, 0))],
        out_shape=[jax.ShapeDtypeStruct((N, H), jnp.float32),
                   jax.ShapeDtypeStruct((NA, N, H), jnp.float32),
                   jax.ShapeDtypeStruct((NG, H), jnp.float32)],
    )(h, batch3, vn, bond, eye)


def _edge_aggr(gtab, gidx, didx, zeros):
    mesh = plsc.VectorSubcoreMesh(core_axis_name="c", subcore_axis_name="s")

    @functools.partial(
        pl.kernel,
        out_type=jax.ShapeDtypeStruct((SC_NC, NP, H), jnp.float32),
        mesh=mesh,
        scratch_types=[
            pltpu.VMEM((S, CH), jnp.int32),
            pltpu.VMEM((S, CH), jnp.int32),
            pltpu.VMEM((2, CH, H), jnp.float32),
            pltpu.VMEM_SHARED((NP, H), jnp.float32),
            pltpu.SemaphoreType.DMA,
        ],
    )
    def k(gtab_hbm, gidx_hbm, didx_hbm, zeros_hbm, out_hbm,
          gidx_v, didx_v, rows_v, aggr_sh, sem):
        cid = lax.axis_index("c")
        sid = lax.axis_index("s")
        wid = sid * SC_NC + cid
        pltpu.sync_copy(zeros_hbm.at[pl.ds(sid * RPS, RPS)],
                        aggr_sh.at[pl.ds(sid * RPS, RPS)])
        pltpu.sync_copy(gidx_hbm.at[wid, 0], gidx_v)
        pltpu.sync_copy(didx_hbm.at[wid, 0], didx_v)
        plsc.subcore_barrier()

        pltpu.async_copy(gtab_hbm.at[gidx_v.at[0]], rows_v.at[0], sem)

        def body(j, carry):
            stg = lax.div(j, S)
            r = lax.rem(j, S)
            par = lax.rem(j, 2)

            @pl.when(jnp.logical_and(r == 0, j > 0))
            def _():
                pltpu.sync_copy(gidx_hbm.at[wid, stg], gidx_v)
                pltpu.sync_copy(didx_hbm.at[wid, stg], didx_v)
                pltpu.async_copy(gtab_hbm.at[gidx_v.at[0]], rows_v.at[par],
                                 sem)

            r2 = lax.rem(j + 1, S)

            @pl.when(jnp.logical_and(r2 != 0, j + 1 < NCH))
            def _():
                pltpu.async_copy(gtab_hbm.at[gidx_v.at[r2]],
                                 rows_v.at[1 - par], sem)

            pltpu.make_async_copy(gtab_hbm.at[gidx_v.at[r]],
                                  rows_v.at[par], sem).wait()
            pltpu.sync_copy(rows_v.at[par], aggr_sh.at[didx_v.at[r]], add=True)
            return carry

        lax.fori_loop(0, NCH, body, 0)
        plsc.subcore_barrier()
        pltpu.sync_copy(aggr_sh.at[pl.ds(sid * RPS, RPS)],
                        out_hbm.at[cid, pl.ds(sid * RPS, RPS)])

    return k(gtab, gidx, didx, zeros)


def _stage_b_body(hl_ref, ag_ref, eps_ref, w1_ref, b1_ref, lg_ref, lb_ref,
                  w2_ref, b2_ref, ng_ref, nb_ref, out_ref, *, last):
    hl = hl_ref[...]
    t = (1.0 + eps_ref[0, 0]) * hl + ag_ref[0] + ag_ref[1]
    t = _dotT(t, w1_ref[...]) + b1_ref[...]
    t = _ln(t, lg_ref[...], lb_ref[...])
    t = _gelu(t)
    t = _dotT(t, w2_ref[...]) + b2_ref[...]
    h = _ln(t, ng_ref[...], nb_ref[...])
    if not last:
        h = _gelu(h)
    out_ref[...] = h + hl


def _stage_b(hl, aggr2, eps_l, w1, b1, lg, lb, w2, b2, ng, nb, last):
    body = functools.partial(_stage_b_body, last=last)
    return pl.pallas_call(
        body,
        grid=(NB,),
        in_specs=[pl.BlockSpec((R, H), lambda i: (i, 0)),
                  pl.BlockSpec((SC_NC, R, H), lambda i: (0, i, 0)),
                  pl.BlockSpec((1, 1), lambda i: (0, 0)),
                  pl.BlockSpec((FF, H), lambda i: (0, 0)),
                  pl.BlockSpec((1, FF), lambda i: (0, 0)),
                  pl.BlockSpec((1, FF), lambda i: (0, 0)),
                  pl.BlockSpec((1, FF), lambda i: (0, 0)),
                  pl.BlockSpec((H, FF), lambda i: (0, 0)),
                  pl.BlockSpec((1, H), lambda i: (0, 0)),
                  pl.BlockSpec((1, H), lambda i: (0, 0)),
                  pl.BlockSpec((1, H), lambda i: (0, 0))],
        out_specs=pl.BlockSpec((R, H), lambda i: (i, 0)),
        out_shape=jax.ShapeDtypeStruct((N, H), jnp.float32),
    )(hl, aggr2, eps_l, w1, b1, lg, lb, w2, b2, ng, nb)


def _vn_body(vp_ref, vn_ref, w1_ref, b1_ref, lg_ref, lb_ref, w2_ref, b2_ref,
             out_ref):
    u = _dotT(vp_ref[...], w1_ref[...]) + b1_ref[...]
    u = _ln(u, lg_ref[...], lb_ref[...])
    u = _gelu(u)
    u = _dotT(u, w2_ref[...]) + b2_ref[...]
    out_ref[...] = vn_ref[...] + u


def _vn_update(vpool, vn, w1, b1, lg, lb, w2, b2):
    return pl.pallas_call(
        _vn_body,
        out_shape=jax.ShapeDtypeStruct((NG, H), jnp.float32),
    )(vpool, vn, w1, b1, lg, lb, w2, b2)


def _segsum_body(h_ref, b_ref, out_ref):
    ids = b_ref[...].reshape(1, R)
    oht = _onehot_t(ids, NG)

    @pl.when(pl.program_id(0) == 0)
    def _():
        out_ref[...] = jnp.zeros((NG, H), jnp.float32)

    out_ref[...] += lax.dot_general(oht, h_ref[...], (((1,), (0,)), ((), ())),
                                    preferred_element_type=jnp.float32)


def _segsum(h, batch3):
    return pl.pallas_call(
        _segsum_body,
        grid=(NB,),
        in_specs=[pl.BlockSpec((R, H), lambda i: (i, 0)),
                  pl.BlockSpec((1, 1, R), lambda i: (i, 0, 0))],
        out_specs=pl.BlockSpec((NG, H), lambda i: (0, 0)),
        out_shape=jax.ShapeDtypeStruct((NG, H), jnp.float32),
    )(h, batch3)


def kernel(x, edge_index, edge_attr, batch, atom_table, vn_table, eps,
           bond_tables, conv_w1, conv_b1, conv_ln_g, conv_ln_b, conv_w2,
           conv_b2, norm_g, norm_b, vn_w1, vn_b1, vn_ln_g, vn_ln_b, vn_w2,
           vn_b2):
    f32 = jnp.float32
    x3 = x.astype(jnp.int32).reshape(NB, 1, R)
    batch3 = batch.astype(jnp.int32).reshape(NB, 1, R)
    atom_pad = jnp.concatenate(
        [atom_table.astype(f32),
         jnp.zeros((128 - atom_table.shape[0], H), f32)], axis=0)
    eye = jnp.eye(NG, dtype=f32)
    gidx = (edge_attr.astype(jnp.int32) * N
            + edge_index[0].astype(jnp.int32)).reshape(SC_NW, NSTG, S, CH)
    didx = edge_index[1].astype(jnp.int32).reshape(SC_NW, NSTG, S, CH)
    zeros = jnp.zeros((NP, H), f32)
    vn = jnp.tile(vn_table.astype(f32), (NG, 1))

    h = _embed(x3, atom_pad)
    for l in range(L):
        hl, gt, vpool = _stage_a(h, batch3, vn, bond_tables[l].astype(f32),
                                 eye, with_vpool=(l < L - 1))
        aggr2 = _edge_aggr(gt.reshape(NA * N, H), gidx, didx, zeros)
        if l < L - 1:
            vn = _vn_update(vpool, vn,
                            vn_w1[l].astype(f32), vn_b1[l].reshape(1, FF),
                            vn_ln_g[l].reshape(1, FF), vn_ln_b[l].reshape(1, FF),
                            vn_w2[l].astype(f32), vn_b2[l].reshape(1, H))
        h = _stage_b(hl, aggr2, eps[l].reshape(1, 1).astype(f32),
                     conv_w1[l].astype(f32), conv_b1[l].reshape(1, FF),
                     conv_ln_g[l].reshape(1, FF), conv_ln_b[l].reshape(1, FF),
                     conv_w2[l].astype(f32), conv_b2[l].reshape(1, H),
                     norm_g[l].reshape(1, H), norm_b[l].reshape(1, H),
                     last=(l == L - 1))
    return _segsum(h, batch3)
